# baseline (speedup 1.0000x reference)
import functools

import jax
import jax.numpy as jnp
from jax import lax
from jax.experimental import pallas as pl
from jax.experimental.pallas import tpu as pltpu
from jax.experimental.pallas import tpu_sc as plsc

B, N, E, NIN, NHID, NOUT = 4, 1024, 16384, 128, 128, 128
TE = 2048
GRID = E // TE
_NSUB = 16
_NCORE = 2
_EPT = E // (_NSUB * _NCORE)
_SLICE = (N * N) // _NSUB


def _sc_build_araw(edge_index):
    mesh = plsc.VectorSubcoreMesh(core_axis_name="c", subcore_axis_name="s")

    @functools.partial(
        pl.kernel,
        out_type=jax.ShapeDtypeStruct((_NCORE * N * N,), jnp.float32),
        mesh=mesh,
        scratch_types=[
            pltpu.VMEM((_EPT,), jnp.int32),
            pltpu.VMEM((_EPT,), jnp.int32),
            pltpu.VMEM((4, 128), jnp.int32),
            pltpu.VMEM((128,), jnp.float32),
            pltpu.VMEM((8192,), jnp.float32),
            pltpu.VMEM_SHARED((N * N,), jnp.float32),
        ],
    )
    def kern(ei_hbm, out_hbm, src_v, dst_v, idx_v, ones_v, zbuf_v, a_sh):
        c = lax.axis_index("c")
        s = lax.axis_index("s")

        def fill_z(i, carry):
            zbuf_v[pl.ds(i * 16, 16)] = jnp.zeros((16,), jnp.float32)
            return carry

        lax.fori_loop(0, 8192 // 16, fill_z, 0)
        for j in range(128 // 16):
            ones_v[pl.ds(j * 16, 16)] = jnp.ones((16,), jnp.float32)

        def zero_slice(j, carry):
            pltpu.sync_copy(zbuf_v, a_sh.at[pl.ds(s * _SLICE + j * 8192, 8192)])
            return carry

        lax.fori_loop(0, _SLICE // 8192, zero_slice, 0)

        base = (s * _NCORE + c) * _EPT
        pltpu.sync_copy(ei_hbm.at[0, pl.ds(base, _EPT)], src_v)
        pltpu.sync_copy(ei_hbm.at[1, pl.ds(base, _EPT)], dst_v)
        for r in range(4):
            for j in range(8):
                i = r * 8 + j
                d = dst_v[pl.ds(i * 16, 16)]
                sv = src_v[pl.ds(i * 16, 16)]
                idx_v[r, pl.ds(j * 16, 16)] = d * N + sv

        plsc.subcore_barrier()

        for r in range(4):
            pltpu.sync_copy(ones_v, a_sh.at[idx_v.at[r]], add=True)

        plsc.subcore_barrier()

        pltpu.sync_copy(
            a_sh.at[pl.ds(s * _SLICE, _SLICE)],
            out_hbm.at[pl.ds(c * (N * N) + s * _SLICE, _SLICE)],
        )

    return kern(edge_index)


def _prep_kernel(ar_ref, x_ref, w1t_ref, b1_ref, an_ref, h_ref):
    a = ar_ref[0].astype(jnp.float32) + ar_ref[1].astype(jnp.float32)
    rows = lax.broadcasted_iota(jnp.int32, (N, N), 0)
    cols = lax.broadcasted_iota(jnp.int32, (N, N), 1)
    a = a + jnp.where(rows == cols, 1.0, 0.0).astype(jnp.float32)
    ones_col = jnp.ones((N, 1), jnp.float32)
    ones_row = jnp.ones((1, N), jnp.float32)
    deg_col = jnp.dot(a, ones_col, preferred_element_type=jnp.float32)
    deg_row = lax.dot_general(ones_row, a, (((1,), (1,)), ((), ())),
                              preferred_element_type=jnp.float32)
    an = a * lax.rsqrt(deg_col) * lax.rsqrt(deg_row)
    an_ref[...] = an
    for b in range(B):
        t = jnp.dot(x_ref[b, 0], w1t_ref[...], preferred_element_type=jnp.float32)
        hb = jnp.dot(an, t, preferred_element_type=jnp.float32) + b1_ref[...]
        h_ref[:, b * NHID:(b + 1) * NHID] = jnp.maximum(hb, 0.0)


def _tc_prep(araw, x, w1t, b1r):
    return pl.pallas_call(
        _prep_kernel,
        out_shape=(
            jax.ShapeDtypeStruct((N, N), jnp.float32),
            jax.ShapeDtypeStruct((N, B * NHID), jnp.float32),
        ),
    )(araw, x, w1t, b1r)


def _main_kernel(min_ref, mout_ref, h_ref, wmrt_ref, wmst_ref, bm_ref,
                 an_ref, w2t_ref, b2_ref, out_ref, acc_ref):
    i = pl.program_id(0)
    mt = min_ref[...].astype(jnp.bfloat16)
    ot = mout_ref[...].astype(jnp.bfloat16)
    h = h_ref[...].astype(jnp.bfloat16)
    recv = jnp.dot(mt, h, preferred_element_type=jnp.float32)
    send = jnp.dot(ot, h, preferred_element_type=jnp.float32)
    parts = []
    for b in range(B):
        rb = recv[:, b * NHID:(b + 1) * NHID]
        sb = send[:, b * NHID:(b + 1) * NHID]
        eb = (jnp.dot(rb, wmrt_ref[...], preferred_element_type=jnp.float32)
              + jnp.dot(sb, wmst_ref[...], preferred_element_type=jnp.float32)
              + bm_ref[...])
        parts.append(jnp.maximum(eb, 0.0))
    e = jnp.concatenate(parts, axis=1).astype(jnp.bfloat16)
    contrib = lax.dot_general(mt, e, (((0,), (0,)), ((), ())),
                              preferred_element_type=jnp.float32)

    @pl.when(i == 0)
    def _():
        acc_ref[...] = contrib

    @pl.when(i > 0)
    def _():
        acc_ref[...] += contrib

    @pl.when(i == GRID - 1)
    def _():
        node = acc_ref[...] * (1.0 / N)
        an = an_ref[...]
        for b in range(B):
            nb = node[:, b * NHID:(b + 1) * NHID]
            t = jnp.dot(nb, w2t_ref[...], preferred_element_type=jnp.float32)
            out_ref[b] = (jnp.dot(an, t, preferred_element_type=jnp.float32)
                          + b2_ref[...])


def _tc_main(m_in, m_out, h, wmrt, wmst, bmr, an, w2t, b2r):
    full = lambda shape: pl.BlockSpec(shape, lambda i: tuple(0 for _ in shape))
    return pl.pallas_call(
        _main_kernel,
        grid=(GRID,),
        in_specs=[
            pl.BlockSpec((TE, N), lambda i: (i, 0)),
            pl.BlockSpec((TE, N), lambda i: (i, 0)),
            full((N, B * NHID)),
            full((NHID, NHID)),
            full((NHID, NHID)),
            full((1, NHID)),
            full((N, N)),
            full((NHID, NHID)),
            full((1, NHID)),
        ],
        out_specs=pl.BlockSpec((B, N, NOUT), lambda i: (0, 0, 0)),
        out_shape=jax.ShapeDtypeStruct((B, N, NOUT), jnp.float32),
        scratch_shapes=[pltpu.VMEM((N, B * NHID), jnp.float32)],
    )(m_in, m_out, h, wmrt, wmst, bmr, an, w2t, b2r)


def kernel(x, edge_index, m_in, m_out, W1, b1, Wm, bm, W2, b2):
    araw = _sc_build_araw(edge_index).reshape(_NCORE, N, N)
    an, h = _tc_prep(araw, x, W1.T, b1.reshape(1, NHID))
    out = _tc_main(m_in, m_out, h,
                   Wm[:, :NHID].T, Wm[:, NHID:].T, bm.reshape(1, NHID),
                   an, W2.T, b2.reshape(1, NHID))
    return out


# speedup vs baseline: 1.0803x; 1.0803x over previous
import functools

import jax
import jax.numpy as jnp
from jax import lax
from jax.experimental import pallas as pl
from jax.experimental.pallas import tpu as pltpu
from jax.experimental.pallas import tpu_sc as plsc

B, N, E, NIN, NHID, NOUT = 4, 1024, 16384, 128, 128, 128
TE = 2048
GRID = E // TE
_NSUB = 16
_NCORE = 2
_EPT = E // _NSUB
_SLICE = (N * N) // _NSUB


def _sc_build_araw(edge_index):
    mesh = plsc.VectorSubcoreMesh(core_axis_name="c", subcore_axis_name="s")

    @functools.partial(
        pl.kernel,
        out_type=jax.ShapeDtypeStruct((N * N,), jnp.float32),
        mesh=mesh,
        scratch_types=[
            pltpu.VMEM((_EPT,), jnp.int32),
            pltpu.VMEM((_EPT,), jnp.int32),
            pltpu.VMEM((8, 128), jnp.int32),
            pltpu.VMEM((128,), jnp.float32),
            pltpu.VMEM((8192,), jnp.float32),
            pltpu.VMEM_SHARED((N * N,), jnp.float32),
            pltpu.SemaphoreType.DMA,
            pltpu.SemaphoreType.DMA,
        ],
    )
    def kern(ei_hbm, out_hbm, src_v, dst_v, idx_v, ones_v, zbuf_v, a_sh,
             sem_z, sem_e):
        c = lax.axis_index("c")
        s = lax.axis_index("s")

        @pl.when(c == 0)
        def _body():
            _kern_body(ei_hbm, out_hbm, src_v, dst_v, idx_v, ones_v, zbuf_v,
                       a_sh, sem_z, sem_e, s)

    def _kern_body(ei_hbm, out_hbm, src_v, dst_v, idx_v, ones_v, zbuf_v,
                   a_sh, sem_z, sem_e, s):
        base = s * _EPT
        cp_s = pltpu.async_copy(ei_hbm.at[0, pl.ds(base, _EPT)], src_v, sem_e)
        cp_d = pltpu.async_copy(ei_hbm.at[1, pl.ds(base, _EPT)], dst_v, sem_e)

        def fill_z(i, carry):
            for u in range(8):
                zbuf_v[pl.ds(i * 128 + u * 16, 16)] = jnp.zeros((16,),
                                                               jnp.float32)
            return carry

        lax.fori_loop(0, 8192 // 128, fill_z, 0)
        for j in range(128 // 16):
            ones_v[pl.ds(j * 16, 16)] = jnp.ones((16,), jnp.float32)

        for j in range(_SLICE // 8192):
            pltpu.async_copy(zbuf_v, a_sh.at[pl.ds(s * _SLICE + j * 8192,
                                                   8192)], sem_z)

        cp_s.wait()
        cp_d.wait()
        for r in range(8):
            for j in range(8):
                i = r * 8 + j
                d = dst_v[pl.ds(i * 16, 16)]
                sv = src_v[pl.ds(i * 16, 16)]
                idx_v[r, pl.ds(j * 16, 16)] = d * N + sv

        for j in range(_SLICE // 8192):
            pltpu.make_async_copy(zbuf_v, a_sh.at[pl.ds(s * _SLICE + j * 8192,
                                                        8192)], sem_z).wait()

        plsc.subcore_barrier()

        for r in range(8):
            pltpu.async_copy(ones_v, a_sh.at[idx_v.at[r]], sem_z, add=True)
        for r in range(8):
            pltpu.make_async_copy(ones_v, a_sh.at[idx_v.at[r]], sem_z).wait()

        plsc.subcore_barrier()

        pltpu.sync_copy(
            a_sh.at[pl.ds(s * _SLICE, _SLICE)],
            out_hbm.at[pl.ds(s * _SLICE, _SLICE)],
        )

    return kern(edge_index)


def _prep_kernel(ar_ref, x_ref, w1t_ref, b1_ref, an_ref, h_ref):
    a = ar_ref[...]
    rows = lax.broadcasted_iota(jnp.int32, (N, N), 0)
    cols = lax.broadcasted_iota(jnp.int32, (N, N), 1)
    a = a + jnp.where(rows == cols, 1.0, 0.0).astype(jnp.float32)
    ones_col = jnp.ones((N, 1), jnp.float32)
    ones_row = jnp.ones((1, N), jnp.float32)
    deg_col = jnp.dot(a, ones_col, preferred_element_type=jnp.float32)
    deg_row = lax.dot_general(ones_row, a, (((1,), (1,)), ((), ())),
                              preferred_element_type=jnp.float32)
    an = a * lax.rsqrt(deg_col) * lax.rsqrt(deg_row)
    an_ref[...] = an
    for b in range(B):
        t = jnp.dot(x_ref[b, 0], w1t_ref[...], preferred_element_type=jnp.float32)
        hb = jnp.dot(an, t, preferred_element_type=jnp.float32) + b1_ref[...]
        h_ref[:, b * NHID:(b + 1) * NHID] = jnp.maximum(hb, 0.0)


def _tc_prep(araw, x, w1t, b1r):
    return pl.pallas_call(
        _prep_kernel,
        out_shape=(
            jax.ShapeDtypeStruct((N, N), jnp.float32),
            jax.ShapeDtypeStruct((N, B * NHID), jnp.float32),
        ),
    )(araw, x, w1t, b1r)


def _main_kernel(min_ref, mout_ref, h_ref, wmrt_ref, wmst_ref, bm_ref,
                 an_ref, w2t_ref, b2_ref, out_ref, acc_ref, e_ref):
    i = pl.program_id(0)
    mt = min_ref[...].astype(jnp.bfloat16)
    ot = mout_ref[...].astype(jnp.bfloat16)
    h = h_ref[...].astype(jnp.bfloat16)
    recv = jnp.dot(mt, h, preferred_element_type=jnp.float32)
    send = jnp.dot(ot, h, preferred_element_type=jnp.float32)
    parts = []
    for b in range(B):
        rb = recv[:, b * NHID:(b + 1) * NHID]
        sb = send[:, b * NHID:(b + 1) * NHID]
        eb = (jnp.dot(rb, wmrt_ref[...], preferred_element_type=jnp.float32)
              + jnp.dot(sb, wmst_ref[...], preferred_element_type=jnp.float32)
              + bm_ref[...])
        parts.append(jnp.maximum(eb, 0.0))
    for b in range(B):
        e_ref[:, b * NHID:(b + 1) * NHID] = parts[b].astype(jnp.bfloat16)
    contrib = lax.dot_general(mt, e_ref[...], (((0,), (0,)), ((), ())),
                              preferred_element_type=jnp.float32)

    @pl.when(i == 0)
    def _():
        acc_ref[...] = contrib

    @pl.when(i > 0)
    def _():
        acc_ref[...] += contrib

    @pl.when(i == GRID - 1)
    def _():
        node = acc_ref[...] * (1.0 / N)
        an = an_ref[...]
        for b in range(B):
            nb = node[:, b * NHID:(b + 1) * NHID]
            t = jnp.dot(nb, w2t_ref[...], preferred_element_type=jnp.float32)
            out_ref[b] = (jnp.dot(an, t, preferred_element_type=jnp.float32)
                          + b2_ref[...])


def _tc_main(m_in, m_out, h, wmrt, wmst, bmr, an, w2t, b2r):
    full = lambda shape: pl.BlockSpec(shape, lambda i: tuple(0 for _ in shape))
    return pl.pallas_call(
        _main_kernel,
        grid=(GRID,),
        in_specs=[
            pl.BlockSpec((TE, N), lambda i: (i, 0)),
            pl.BlockSpec((TE, N), lambda i: (i, 0)),
            full((N, B * NHID)),
            full((NHID, NHID)),
            full((NHID, NHID)),
            full((1, NHID)),
            full((N, N)),
            full((NHID, NHID)),
            full((1, NHID)),
        ],
        out_specs=pl.BlockSpec((B, N, NOUT), lambda i: (0, 0, 0)),
        out_shape=jax.ShapeDtypeStruct((B, N, NOUT), jnp.float32),
        scratch_shapes=[pltpu.VMEM((N, B * NHID), jnp.float32),
                        pltpu.VMEM((TE, B * NHID), jnp.bfloat16)],
    )(m_in, m_out, h, wmrt, wmst, bmr, an, w2t, b2r)


def kernel(x, edge_index, m_in, m_out, W1, b1, Wm, bm, W2, b2):
    araw = _sc_build_araw(edge_index).reshape(N, N)
    an, h = _tc_prep(araw, x, W1.T, b1.reshape(1, NHID))
    out = _tc_main(m_in, m_out, h,
                   Wm[:, :NHID].T, Wm[:, NHID:].T, bm.reshape(1, NHID),
                   an, W2.T, b2.reshape(1, NHID))
    return out


# speedup vs baseline: 1.0932x; 1.0120x over previous
import functools

import jax
import jax.numpy as jnp
from jax import lax
from jax.experimental import pallas as pl
from jax.experimental.pallas import tpu as pltpu
from jax.experimental.pallas import tpu_sc as plsc

B, N, E, NIN, NHID, NOUT = 4, 1024, 16384, 128, 128, 128
TE = 2048
GRID = E // TE
_NSUB = 16
_NCORE = 2
_EPT = E // _NSUB
_SLICE = (N * N) // _NSUB


def _sc_build_araw(edge_index):
    mesh = plsc.VectorSubcoreMesh(core_axis_name="c", subcore_axis_name="s")

    @functools.partial(
        pl.kernel,
        out_type=jax.ShapeDtypeStruct((N * N,), jnp.float32),
        mesh=mesh,
        scratch_types=[
            pltpu.VMEM((_EPT,), jnp.int32),
            pltpu.VMEM((_EPT,), jnp.int32),
            pltpu.VMEM((8, 128), jnp.int32),
            pltpu.VMEM((128,), jnp.float32),
            pltpu.VMEM((8192,), jnp.float32),
            pltpu.VMEM_SHARED((N * N,), jnp.float32),
            pltpu.SemaphoreType.DMA,
            pltpu.SemaphoreType.DMA,
        ],
    )
    def kern(ei_hbm, out_hbm, src_v, dst_v, idx_v, ones_v, zbuf_v, a_sh,
             sem_z, sem_e):
        c = lax.axis_index("c")
        s = lax.axis_index("s")

        @pl.when(c == 0)
        def _body():
            _kern_body(ei_hbm, out_hbm, src_v, dst_v, idx_v, ones_v, zbuf_v,
                       a_sh, sem_z, sem_e, s)

    def _kern_body(ei_hbm, out_hbm, src_v, dst_v, idx_v, ones_v, zbuf_v,
                   a_sh, sem_z, sem_e, s):
        base = s * _EPT
        cp_s = pltpu.async_copy(ei_hbm.at[0, pl.ds(base, _EPT)], src_v, sem_e)
        cp_d = pltpu.async_copy(ei_hbm.at[1, pl.ds(base, _EPT)], dst_v, sem_e)

        def fill_z(i, carry):
            for u in range(8):
                zbuf_v[pl.ds(i * 128 + u * 16, 16)] = jnp.zeros((16,),
                                                               jnp.float32)
            return carry

        lax.fori_loop(0, 8192 // 128, fill_z, 0)
        for j in range(128 // 16):
            ones_v[pl.ds(j * 16, 16)] = jnp.ones((16,), jnp.float32)

        for j in range(_SLICE // 8192):
            pltpu.async_copy(zbuf_v, a_sh.at[pl.ds(s * _SLICE + j * 8192,
                                                   8192)], sem_z)

        cp_s.wait()
        cp_d.wait()
        for r in range(8):
            for j in range(8):
                i = r * 8 + j
                d = dst_v[pl.ds(i * 16, 16)]
                sv = src_v[pl.ds(i * 16, 16)]
                idx_v[r, pl.ds(j * 16, 16)] = d * N + sv

        for j in range(_SLICE // 8192):
            pltpu.make_async_copy(zbuf_v, a_sh.at[pl.ds(s * _SLICE + j * 8192,
                                                        8192)], sem_z).wait()

        plsc.subcore_barrier()

        for r in range(8):
            pltpu.async_copy(ones_v, a_sh.at[idx_v.at[r]], sem_z, add=True)
        for r in range(8):
            pltpu.make_async_copy(ones_v, a_sh.at[idx_v.at[r]], sem_z).wait()

        plsc.subcore_barrier()

        pltpu.sync_copy(
            a_sh.at[pl.ds(s * _SLICE, _SLICE)],
            out_hbm.at[pl.ds(s * _SLICE, _SLICE)],
        )

    return kern(edge_index)


def _prep_kernel(ar_ref, x_ref, w1t_ref, b1_ref, an_ref, h_ref):
    a = ar_ref[...]
    rows = lax.broadcasted_iota(jnp.int32, (N, N), 0)
    cols = lax.broadcasted_iota(jnp.int32, (N, N), 1)
    a = a + jnp.where(rows == cols, 1.0, 0.0).astype(jnp.float32)
    ones_col = jnp.ones((N, 1), jnp.float32)
    ones_row = jnp.ones((1, N), jnp.float32)
    deg_col = jnp.dot(a, ones_col, preferred_element_type=jnp.float32)
    deg_row = lax.dot_general(ones_row, a, (((1,), (1,)), ((), ())),
                              preferred_element_type=jnp.float32)
    an = a * lax.rsqrt(deg_col) * lax.rsqrt(deg_row)
    an_ref[...] = an.astype(jnp.bfloat16)
    for b in range(B):
        t = jnp.dot(x_ref[b, 0], w1t_ref[...], preferred_element_type=jnp.float32)
        hb = jnp.dot(an, t, preferred_element_type=jnp.float32) + b1_ref[...]
        h_ref[:, b * NHID:(b + 1) * NHID] = jnp.maximum(hb, 0.0).astype(jnp.bfloat16)


def _tc_prep(araw, x, w1t, b1r):
    return pl.pallas_call(
        _prep_kernel,
        out_shape=(
            jax.ShapeDtypeStruct((N, N), jnp.bfloat16),
            jax.ShapeDtypeStruct((N, B * NHID), jnp.bfloat16),
        ),
    )(araw, x, w1t, b1r)


def _main_kernel(min_ref, mout_ref, h_ref, wmrt_ref, wmst_ref, bm_ref,
                 an_ref, w2t_ref, b2_ref, out_ref, acc_ref, e_ref):
    i = pl.program_id(0)
    mt = min_ref[...].astype(jnp.bfloat16)
    ot = mout_ref[...].astype(jnp.bfloat16)
    h = h_ref[...]
    recv = jnp.dot(mt, h, preferred_element_type=jnp.float32)
    send = jnp.dot(ot, h, preferred_element_type=jnp.float32)
    parts = []
    for b in range(B):
        rb = recv[:, b * NHID:(b + 1) * NHID]
        sb = send[:, b * NHID:(b + 1) * NHID]
        eb = (jnp.dot(rb, wmrt_ref[...], preferred_element_type=jnp.float32)
              + jnp.dot(sb, wmst_ref[...], preferred_element_type=jnp.float32)
              + bm_ref[...])
        parts.append(jnp.maximum(eb, 0.0))
    for b in range(B):
        e_ref[:, b * NHID:(b + 1) * NHID] = parts[b].astype(jnp.bfloat16)
    contrib = lax.dot_general(mt, e_ref[...], (((0,), (0,)), ((), ())),
                              preferred_element_type=jnp.float32)

    @pl.when(i == 0)
    def _():
        acc_ref[...] = contrib

    @pl.when(i > 0)
    def _():
        acc_ref[...] += contrib

    @pl.when(i == GRID - 1)
    def _():
        node = acc_ref[...] * (1.0 / N)
        an = an_ref[...]
        for b in range(B):
            nb = node[:, b * NHID:(b + 1) * NHID]
            t = jnp.dot(nb, w2t_ref[...], preferred_element_type=jnp.float32)
            out_ref[b] = (jnp.dot(an, t.astype(jnp.bfloat16),
                                  preferred_element_type=jnp.float32)
                          + b2_ref[...])


def _tc_main(m_in, m_out, h, wmrt, wmst, bmr, an, w2t, b2r):
    full = lambda shape: pl.BlockSpec(shape, lambda i: tuple(0 for _ in shape))
    return pl.pallas_call(
        _main_kernel,
        grid=(GRID,),
        in_specs=[
            pl.BlockSpec((TE, N), lambda i: (i, 0)),
            pl.BlockSpec((TE, N), lambda i: (i, 0)),
            full((N, B * NHID)),
            full((NHID, NHID)),
            full((NHID, NHID)),
            full((1, NHID)),
            full((N, N)),
            full((NHID, NHID)),
            full((1, NHID)),
        ],
        out_specs=pl.BlockSpec((B, N, NOUT), lambda i: (0, 0, 0)),
        out_shape=jax.ShapeDtypeStruct((B, N, NOUT), jnp.float32),
        scratch_shapes=[pltpu.VMEM((N, B * NHID), jnp.float32),
                        pltpu.VMEM((TE, B * NHID), jnp.bfloat16)],
    )(m_in, m_out, h, wmrt, wmst, bmr, an, w2t, b2r)


def kernel(x, edge_index, m_in, m_out, W1, b1, Wm, bm, W2, b2):
    araw = _sc_build_araw(edge_index).reshape(N, N)
    an, h = _tc_prep(araw, x, W1.T, b1.reshape(1, NHID))
    out = _tc_main(m_in, m_out, h,
                   Wm[:, :NHID].T, Wm[:, NHID:].T, bm.reshape(1, NHID),
                   an, W2.T, b2.reshape(1, NHID))
    return out


# speedup vs baseline: 1.1380x; 1.0409x over previous
import functools

import jax
import jax.numpy as jnp
from jax import lax
from jax.experimental import pallas as pl
from jax.experimental.pallas import tpu as pltpu
from jax.experimental.pallas import tpu_sc as plsc

B, N, E, NIN, NHID, NOUT = 4, 1024, 16384, 128, 128, 128
TE = 2048
GRID = E // TE
_NSUB = 16
_NCORE = 2
_EPT = E // _NSUB
_SLICE = (N * N) // _NSUB


def _sc_build_araw(edge_index):
    mesh = plsc.VectorSubcoreMesh(core_axis_name="c", subcore_axis_name="s")

    @functools.partial(
        pl.kernel,
        out_type=jax.ShapeDtypeStruct((N * N,), jnp.float32),
        mesh=mesh,
        scratch_types=[
            pltpu.VMEM((_EPT,), jnp.int32),
            pltpu.VMEM((_EPT,), jnp.int32),
            pltpu.VMEM((8, 128), jnp.int32),
            pltpu.VMEM((128,), jnp.float32),
            pltpu.VMEM((8192,), jnp.float32),
            pltpu.VMEM_SHARED((N * N,), jnp.float32),
            pltpu.SemaphoreType.DMA,
            pltpu.SemaphoreType.DMA,
        ],
    )
    def kern(ei_hbm, out_hbm, src_v, dst_v, idx_v, ones_v, zbuf_v, a_sh,
             sem_z, sem_e):
        c = lax.axis_index("c")
        s = lax.axis_index("s")

        @pl.when(c == 0)
        def _body():
            _kern_body(ei_hbm, out_hbm, src_v, dst_v, idx_v, ones_v, zbuf_v,
                       a_sh, sem_z, sem_e, s)

    def _kern_body(ei_hbm, out_hbm, src_v, dst_v, idx_v, ones_v, zbuf_v,
                   a_sh, sem_z, sem_e, s):
        base = s * _EPT
        cp_s = pltpu.async_copy(ei_hbm.at[0, pl.ds(base, _EPT)], src_v, sem_e)
        cp_d = pltpu.async_copy(ei_hbm.at[1, pl.ds(base, _EPT)], dst_v, sem_e)

        def fill_z(i, carry):
            for u in range(8):
                zbuf_v[pl.ds(i * 128 + u * 16, 16)] = jnp.zeros((16,),
                                                               jnp.float32)
            return carry

        lax.fori_loop(0, 8192 // 128, fill_z, 0)
        for j in range(128 // 16):
            ones_v[pl.ds(j * 16, 16)] = jnp.ones((16,), jnp.float32)

        for j in range(_SLICE // 8192):
            pltpu.async_copy(zbuf_v, a_sh.at[pl.ds(s * _SLICE + j * 8192,
                                                   8192)], sem_z)

        cp_s.wait()
        cp_d.wait()
        for r in range(8):
            for j in range(8):
                i = r * 8 + j
                d = dst_v[pl.ds(i * 16, 16)]
                sv = src_v[pl.ds(i * 16, 16)]
                idx_v[r, pl.ds(j * 16, 16)] = ((sv >> 7) * (N * 128)
                                               + d * 128 + (sv & 127))

        for j in range(_SLICE // 8192):
            pltpu.make_async_copy(zbuf_v, a_sh.at[pl.ds(s * _SLICE + j * 8192,
                                                        8192)], sem_z).wait()

        plsc.subcore_barrier()

        for r in range(8):
            pltpu.async_copy(ones_v, a_sh.at[idx_v.at[r]], sem_z, add=True)
        for r in range(8):
            pltpu.make_async_copy(ones_v, a_sh.at[idx_v.at[r]], sem_z).wait()

        plsc.subcore_barrier()

        pltpu.sync_copy(
            a_sh.at[pl.ds(s * _SLICE, _SLICE)],
            out_hbm.at[pl.ds(s * _SLICE, _SLICE)],
        )

    return kern(edge_index)


def _prep_kernel(ar_ref, x_ref, w1t_ref, b1_ref, an_ref, h_ref):
    a = jnp.concatenate([ar_ref[k] for k in range(N // 128)], axis=1)
    rows = lax.broadcasted_iota(jnp.int32, (N, N), 0)
    cols = lax.broadcasted_iota(jnp.int32, (N, N), 1)
    a = a + jnp.where(rows == cols, 1.0, 0.0).astype(jnp.float32)
    ones_col = jnp.ones((N, 1), jnp.float32)
    ones_row = jnp.ones((1, N), jnp.float32)
    deg_col = jnp.dot(a, ones_col, preferred_element_type=jnp.float32)
    deg_row = lax.dot_general(ones_row, a, (((1,), (1,)), ((), ())),
                              preferred_element_type=jnp.float32)
    an = a * lax.rsqrt(deg_col) * lax.rsqrt(deg_row)
    an_ref[...] = an.astype(jnp.bfloat16)
    for b in range(B):
        t = jnp.dot(x_ref[b, 0], w1t_ref[...], preferred_element_type=jnp.float32)
        hb = jnp.dot(an, t, preferred_element_type=jnp.float32) + b1_ref[...]
        h_ref[:, b * NHID:(b + 1) * NHID] = jnp.maximum(hb, 0.0).astype(jnp.bfloat16)


def _tc_prep(araw, x, w1t, b1r):
    return pl.pallas_call(
        _prep_kernel,
        out_shape=(
            jax.ShapeDtypeStruct((N, N), jnp.bfloat16),
            jax.ShapeDtypeStruct((N, B * NHID), jnp.bfloat16),
        ),
    )(araw, x, w1t, b1r)


def _main_kernel(min_ref, mout_ref, h_ref, wmrt_ref, wmst_ref, bm_ref,
                 an_ref, w2t_ref, b2_ref, out_ref, acc_ref, e_ref):
    i = pl.program_id(0)
    mt = min_ref[...].astype(jnp.bfloat16)
    ot = mout_ref[...].astype(jnp.bfloat16)
    h = h_ref[...]
    recv = jnp.dot(mt, h, preferred_element_type=jnp.float32)
    send = jnp.dot(ot, h, preferred_element_type=jnp.float32)
    parts = []
    for b in range(B):
        rb = recv[:, b * NHID:(b + 1) * NHID]
        sb = send[:, b * NHID:(b + 1) * NHID]
        eb = (jnp.dot(rb, wmrt_ref[...], preferred_element_type=jnp.float32)
              + jnp.dot(sb, wmst_ref[...], preferred_element_type=jnp.float32)
              + bm_ref[...])
        parts.append(jnp.maximum(eb, 0.0))
    for b in range(B):
        e_ref[:, b * NHID:(b + 1) * NHID] = parts[b].astype(jnp.bfloat16)
    contrib = lax.dot_general(mt, e_ref[...], (((0,), (0,)), ((), ())),
                              preferred_element_type=jnp.float32)

    @pl.when(i == 0)
    def _():
        acc_ref[...] = contrib

    @pl.when(i > 0)
    def _():
        acc_ref[...] += contrib

    @pl.when(i == GRID - 1)
    def _():
        node = acc_ref[...] * (1.0 / N)
        an = an_ref[...]
        for b in range(B):
            nb = node[:, b * NHID:(b + 1) * NHID]
            t = jnp.dot(nb, w2t_ref[...], preferred_element_type=jnp.float32)
            out_ref[b] = (jnp.dot(an, t.astype(jnp.bfloat16),
                                  preferred_element_type=jnp.float32)
                          + b2_ref[...])


def _tc_main(m_in, m_out, h, wmrt, wmst, bmr, an, w2t, b2r):
    full = lambda shape: pl.BlockSpec(shape, lambda i: tuple(0 for _ in shape))
    return pl.pallas_call(
        _main_kernel,
        grid=(GRID,),
        in_specs=[
            pl.BlockSpec((TE, N), lambda i: (i, 0)),
            pl.BlockSpec((TE, N), lambda i: (i, 0)),
            full((N, B * NHID)),
            full((NHID, NHID)),
            full((NHID, NHID)),
            full((1, NHID)),
            full((N, N)),
            full((NHID, NHID)),
            full((1, NHID)),
        ],
        out_specs=pl.BlockSpec((B, N, NOUT), lambda i: (0, 0, 0)),
        out_shape=jax.ShapeDtypeStruct((B, N, NOUT), jnp.float32),
        scratch_shapes=[pltpu.VMEM((N, B * NHID), jnp.float32),
                        pltpu.VMEM((TE, B * NHID), jnp.bfloat16)],
    )(m_in, m_out, h, wmrt, wmst, bmr, an, w2t, b2r)


def kernel(x, edge_index, m_in, m_out, W1, b1, Wm, bm, W2, b2):
    araw = _sc_build_araw(edge_index).reshape(N // 128, N, 128)
    an, h = _tc_prep(araw, x, W1.T, b1.reshape(1, NHID))
    out = _tc_main(m_in, m_out, h,
                   Wm[:, :NHID].T, Wm[:, NHID:].T, bm.reshape(1, NHID),
                   an, W2.T, b2.reshape(1, NHID))
    return out
